# Initial kernel scaffold; baseline (speedup 1.0000x reference)
import functools

import jax
import jax.numpy as jnp
from jax import lax
from jax.experimental import pallas as pl
from jax.experimental.pallas import tpu as pltpu
from jax.experimental.pallas import tpu_sc as plsc

_N_NODES = 10000
_N_EDGES = 320000
_D = 128
_NC = 2
_NS = 16
_NW = _NC * _NS
_EPT = _N_EDGES // _NW
_CHUNK = 125
_NCHUNK = _EPT // _CHUNK
_ROWS_PT = _N_NODES // _NS
_CW = 16

_mesh = plsc.VectorSubcoreMesh(core_axis_name="c", subcore_axis_name="s")


@functools.partial(
    pl.kernel,
    out_type=[
        jax.ShapeDtypeStruct((_NC, _N_NODES, _D), jnp.float32),
        jax.ShapeDtypeStruct((_NC, _N_NODES, _CW), jnp.float32),
    ],
    mesh=_mesh,
    scratch_types=[
        pltpu.VMEM_SHARED((_N_NODES, _D), jnp.float32),
        pltpu.VMEM_SHARED((_N_NODES, _CW), jnp.float32),
        pltpu.VMEM((_NCHUNK, _CHUNK), jnp.int32),
        pltpu.VMEM((_CHUNK, _D), jnp.float32),
        pltpu.VMEM((_CHUNK, _CW), jnp.float32),
    ],
)
def _sc_aggregate(edges_hbm, idx_hbm, zeros_hbm, zeros_cw_hbm, ones_hbm,
                  sums_out, cnts_out,
                  accum, cnt_sh, idx_v, ebuf, ones_v):
    cid = lax.axis_index("c")
    sid = lax.axis_index("s")
    wid = cid * _NS + sid
    pltpu.sync_copy(zeros_hbm, accum.at[pl.ds(sid * _ROWS_PT, _ROWS_PT)])
    pltpu.sync_copy(zeros_cw_hbm, cnt_sh.at[pl.ds(sid * _ROWS_PT, _ROWS_PT)])
    pltpu.sync_copy(ones_hbm, ones_v)
    pltpu.sync_copy(idx_hbm.at[wid], idx_v)
    plsc.subcore_barrier()

    base = wid * _EPT

    def body(j, carry):
        pltpu.sync_copy(edges_hbm.at[pl.ds(base + j * _CHUNK, _CHUNK)], ebuf)
        row = idx_v.at[j]
        pltpu.sync_copy(ebuf, accum.at[row], add=True)
        pltpu.sync_copy(ones_v, cnt_sh.at[row], add=True)
        return carry

    lax.fori_loop(0, _NCHUNK, body, 0)
    plsc.subcore_barrier()

    pltpu.sync_copy(accum.at[pl.ds(sid * _ROWS_PT, _ROWS_PT)],
                    sums_out.at[cid, pl.ds(sid * _ROWS_PT, _ROWS_PT)])
    pltpu.sync_copy(cnt_sh.at[pl.ds(sid * _ROWS_PT, _ROWS_PT)],
                    cnts_out.at[cid, pl.ds(sid * _ROWS_PT, _ROWS_PT)])


_BLK = 1000


def _tc_body(s_ref, c_ref, w_ref, b_ref, o_ref):
    s = s_ref[0] + s_ref[1]
    cnt = c_ref[0, :, 0:1] + c_ref[1, :, 0:1]
    mean = s / jnp.maximum(cnt, 1.0)
    acc = lax.dot_general(mean, w_ref[...], (((1,), (1,)), ((), ())),
                          preferred_element_type=jnp.float32)
    o_ref[...] = jnp.maximum(acc + b_ref[...], 0.0)


_tc_update = pl.pallas_call(
    _tc_body,
    grid=(_N_NODES // _BLK,),
    in_specs=[
        pl.BlockSpec((_NC, _BLK, _D), lambda i: (0, i, 0)),
        pl.BlockSpec((_NC, _BLK, _CW), lambda i: (0, i, 0)),
        pl.BlockSpec((_D, _D), lambda i: (0, 0)),
        pl.BlockSpec((1, _D), lambda i: (0, 0)),
    ],
    out_specs=pl.BlockSpec((_BLK, _D), lambda i: (i, 0)),
    out_shape=jax.ShapeDtypeStruct((_N_NODES, _D), jnp.float32),
)


def kernel(edge_data, edge_index, W, b):
    dst = edge_index[1].reshape(_NW, _NCHUNK, _CHUNK)
    zeros = jnp.zeros((_ROWS_PT, _D), jnp.float32)
    zeros_cw = jnp.zeros((_ROWS_PT, _CW), jnp.float32)
    ones = jnp.ones((_CHUNK, _CW), jnp.float32)
    sums, cnts = _sc_aggregate(edge_data, dst, zeros, zeros_cw, ones)
    return _tc_update(sums, cnts, W, b.reshape(1, _D))


# speedup vs baseline: 2.7196x; 2.7196x over previous
import functools

import jax
import jax.numpy as jnp
from jax import lax
from jax.experimental import pallas as pl
from jax.experimental.pallas import tpu as pltpu
from jax.experimental.pallas import tpu_sc as plsc

_N_NODES = 10000
_N_EDGES = 320000
_D = 128
_NC = 2
_NS = 16
_NW = _NC * _NS
_CHUNK = 128
_NCHUNKS = _N_EDGES // _CHUNK
_GRP = 8
_NGROUPS = (_NCHUNKS + _GRP - 1) // _GRP
_NGMAX = (_NGROUPS + _NW - 1) // _NW
_NPAD = 10240
_ROWS_PT = _NPAD // _NS
_NHIST = 16384
_L = 16

_mesh = plsc.VectorSubcoreMesh(
    core_axis_name="c", subcore_axis_name="s", num_cores=_NC, num_subcores=_NS)


@functools.partial(
    pl.kernel,
    out_type=[
        jax.ShapeDtypeStruct((_NC, _NPAD, _D), jnp.float32),
        jax.ShapeDtypeStruct((_NW * _NHIST,), jnp.float32),
    ],
    mesh=_mesh,
    scratch_types=[
        pltpu.VMEM_SHARED((_NPAD, _D), jnp.float32),
        pltpu.VMEM((_GRP, _CHUNK), jnp.int32),
        pltpu.VMEM((_CHUNK, _D), jnp.float32),
        pltpu.VMEM((_NHIST,), jnp.float32),
    ],
    compiler_params=pltpu.CompilerParams(needs_layout_passes=False),
)
def _sc_aggregate(edges_hbm, idx_hbm, zeros_hbm, zeros_h_hbm,
                  sums_out, cnts_out,
                  accum, idx_blk, ebuf, cnt_v):
    cid = lax.axis_index("c")
    sid = lax.axis_index("s")
    wid = cid * _NS + sid
    rbase = pl.multiple_of(sid * _ROWS_PT, _ROWS_PT)
    pltpu.sync_copy(zeros_hbm, accum.at[pl.ds(rbase, _ROWS_PT)])
    pltpu.sync_copy(zeros_h_hbm, cnt_v)
    plsc.subcore_barrier()

    ones16 = jnp.full((_L,), 1.0, jnp.float32)

    def group_body(j, carry):
        g = wid + _NW * j

        @pl.when(g < _NGROUPS)
        def _process_group():
            pltpu.sync_copy(idx_hbm.at[g], idx_blk)
            for r in range(_GRP):
                q = g * _GRP + r

                @pl.when(q < _NCHUNKS)
                def _process_chunk():
                    ebase = pl.multiple_of(q * _CHUNK, _CHUNK)
                    pltpu.sync_copy(edges_hbm.at[pl.ds(ebase, _CHUNK)], ebuf)
                    pltpu.sync_copy(ebuf, accum.at[idx_blk.at[r]], add=True)
                    for k in range(_CHUNK // _L):
                        iv = idx_blk[r, pl.ds(k * _L, _L)]
                        plsc.addupdate_scatter(cnt_v, [iv], ones16)

        return carry

    lax.fori_loop(0, _NGMAX, group_body, 0)
    plsc.subcore_barrier()

    pltpu.sync_copy(accum.at[pl.ds(rbase, _ROWS_PT)],
                    sums_out.at[cid, pl.ds(rbase, _ROWS_PT)])
    hbase = pl.multiple_of(wid * _NHIST, _NHIST)
    pltpu.sync_copy(cnt_v, cnts_out.at[pl.ds(hbase, _NHIST)])


_BLK = 1024
_HROWS = _BLK // _D


def _tc_body(s_ref, c_ref, w_ref, b_ref, o_ref):
    s = s_ref[0] + s_ref[1]
    cnt = jnp.sum(c_ref[...], axis=0)
    mean = s / jnp.maximum(cnt, 1.0)
    acc = lax.dot_general(mean, w_ref[...], (((1,), (1,)), ((), ())),
                          preferred_element_type=jnp.float32)
    o_ref[...] = jnp.maximum(acc + b_ref[...], 0.0)


_tc_update = pl.pallas_call(
    _tc_body,
    grid=((_N_NODES + _BLK - 1) // _BLK,),
    in_specs=[
        pl.BlockSpec((_NC, _BLK, _D), lambda i: (0, i, 0)),
        pl.BlockSpec((_NW, _BLK, 1), lambda i: (0, i, 0)),
        pl.BlockSpec((_D, _D), lambda i: (0, 0)),
        pl.BlockSpec((1, _D), lambda i: (0, 0)),
    ],
    out_specs=pl.BlockSpec((_BLK, _D), lambda i: (i, 0)),
    out_shape=jax.ShapeDtypeStruct((_N_NODES, _D), jnp.float32),
)


def kernel(edge_data, edge_index, W, b):
    dst = edge_index[1]
    dst_pad = jnp.pad(dst, (0, _NGROUPS * _GRP * _CHUNK - _N_EDGES))
    idx3 = dst_pad.reshape(_NGROUPS, _GRP, _CHUNK)
    zeros = jnp.zeros((_ROWS_PT, _D), jnp.float32)
    zeros_h = jnp.zeros((_NHIST,), jnp.float32)
    sums, cnts = _sc_aggregate(edge_data, idx3, zeros, zeros_h)
    cnts3 = cnts.reshape(_NW, _NHIST, 1)
    return _tc_update(sums, cnts3, W, b.reshape(1, _D))


# speedup vs baseline: 3.7972x; 1.3962x over previous
import functools

import jax
import jax.numpy as jnp
from jax import lax
from jax.experimental import pallas as pl
from jax.experimental.pallas import tpu as pltpu
from jax.experimental.pallas import tpu_sc as plsc

_N_NODES = 10000
_N_EDGES = 320000
_D = 128
_NC = 2
_NS = 16
_NW = _NC * _NS
_CHUNK = 128
_NCHUNKS = _N_EDGES // _CHUNK
_GRP = 8
_NGROUPS = (_NCHUNKS + _GRP - 1) // _GRP
_NGMAX = (_NGROUPS + _NW - 1) // _NW
_NPAD = 10240
_ROWS_PT = _NPAD // _NS
_NHIST = 10240
_L = 16

_mesh = plsc.VectorSubcoreMesh(
    core_axis_name="c", subcore_axis_name="s", num_cores=_NC, num_subcores=_NS)


@functools.partial(
    pl.kernel,
    out_type=[
        jax.ShapeDtypeStruct((_NC, _NPAD, _D), jnp.float32),
        jax.ShapeDtypeStruct((_NW * _NHIST,), jnp.float32),
    ],
    mesh=_mesh,
    scratch_types=[
        pltpu.VMEM_SHARED((_NPAD, _D), jnp.float32),
        pltpu.VMEM((_GRP, _CHUNK), jnp.int32),
        pltpu.VMEM((_CHUNK, _D), jnp.float32),
        pltpu.VMEM((_CHUNK, _D), jnp.float32),
        pltpu.VMEM((_NHIST,), jnp.float32),
        pltpu.SemaphoreType.DMA,
        pltpu.SemaphoreType.DMA,
    ],
    compiler_params=pltpu.CompilerParams(needs_layout_passes=False),
)
def _sc_aggregate(edges_hbm, idx_hbm, zeros_hbm, zeros_h_hbm,
                  sums_out, cnts_out,
                  accum, idx_blk, ebuf0, ebuf1, cnt_v, sem0, sem1):
    cid = lax.axis_index("c")
    sid = lax.axis_index("s")
    wid = cid * _NS + sid
    rbase = pl.multiple_of(sid * _ROWS_PT, _ROWS_PT)
    pltpu.sync_copy(zeros_hbm, accum.at[pl.ds(rbase, _ROWS_PT)])
    pltpu.sync_copy(zeros_h_hbm, cnt_v)
    plsc.subcore_barrier()

    ones16 = jnp.full((_L,), 1.0, jnp.float32)

    def group_body(j, carry):
        g = wid + _NW * j

        @pl.when(g < _NGROUPS)
        def _process_group():
            pltpu.sync_copy(idx_hbm.at[g], idx_blk)
            bufs = (ebuf0, ebuf1)
            sems = (sem0, sem1)

            @pl.when(g * _GRP < _NCHUNKS)
            def _fire_first():
                ebase = pl.multiple_of(g * _GRP * _CHUNK, _CHUNK)
                pltpu.async_copy(edges_hbm.at[pl.ds(ebase, _CHUNK)],
                                 ebuf0, sem0)

            for r in range(_GRP):
                q = g * _GRP + r
                buf, sem = bufs[r % 2], sems[r % 2]
                nbuf, nsem = bufs[(r + 1) % 2], sems[(r + 1) % 2]

                if r + 1 < _GRP:
                    @pl.when(q + 1 < _NCHUNKS)
                    def _fire_next(q=q, nbuf=nbuf, nsem=nsem):
                        ebase = pl.multiple_of((q + 1) * _CHUNK, _CHUNK)
                        pltpu.async_copy(edges_hbm.at[pl.ds(ebase, _CHUNK)],
                                         nbuf, nsem)

                @pl.when(q < _NCHUNKS)
                def _process_chunk(q=q, r=r, buf=buf, sem=sem):
                    ebase = pl.multiple_of(q * _CHUNK, _CHUNK)
                    pltpu.make_async_copy(edges_hbm.at[pl.ds(ebase, _CHUNK)],
                                          buf, sem).wait()
                    pltpu.sync_copy(buf, accum.at[idx_blk.at[r]], add=True)
                    for k in range(_CHUNK // _L):
                        iv = idx_blk[r, pl.ds(k * _L, _L)]
                        plsc.addupdate_scatter(cnt_v, [iv], ones16)

        return carry

    lax.fori_loop(0, _NGMAX, group_body, 0)
    plsc.subcore_barrier()

    pltpu.sync_copy(accum.at[pl.ds(rbase, _ROWS_PT)],
                    sums_out.at[cid, pl.ds(rbase, _ROWS_PT)])
    hbase = pl.multiple_of(wid * _NHIST, _NHIST)
    pltpu.sync_copy(cnt_v, cnts_out.at[pl.ds(hbase, _NHIST)])


_BLK = 1024
_HROWS = _BLK // _D


def _tc_body(s_ref, c_ref, w_ref, b_ref, o_ref):
    s = s_ref[0] + s_ref[1]
    cnt = jnp.sum(c_ref[...], axis=0)
    mean = s / jnp.maximum(cnt, 1.0)
    acc = lax.dot_general(mean, w_ref[...], (((1,), (1,)), ((), ())),
                          preferred_element_type=jnp.float32)
    o_ref[...] = jnp.maximum(acc + b_ref[...], 0.0)


_tc_update = pl.pallas_call(
    _tc_body,
    grid=((_N_NODES + _BLK - 1) // _BLK,),
    in_specs=[
        pl.BlockSpec((_NC, _BLK, _D), lambda i: (0, i, 0)),
        pl.BlockSpec((_NW, _BLK, 1), lambda i: (0, i, 0)),
        pl.BlockSpec((_D, _D), lambda i: (0, 0)),
        pl.BlockSpec((1, _D), lambda i: (0, 0)),
    ],
    out_specs=pl.BlockSpec((_BLK, _D), lambda i: (i, 0)),
    out_shape=jax.ShapeDtypeStruct((_N_NODES, _D), jnp.float32),
)


def kernel(edge_data, edge_index, W, b):
    dst = edge_index[1]
    dst_pad = jnp.pad(dst, (0, _NGROUPS * _GRP * _CHUNK - _N_EDGES))
    idx3 = dst_pad.reshape(_NGROUPS, _GRP, _CHUNK)
    zeros = jnp.zeros((_ROWS_PT, _D), jnp.float32)
    zeros_h = jnp.zeros((_NHIST,), jnp.float32)
    sums, cnts = _sc_aggregate(edge_data, idx3, zeros, zeros_h)
    cnts3 = cnts.reshape(_NW, _NHIST, 1)
    return _tc_update(sums, cnts3, W, b.reshape(1, _D))


# speedup vs baseline: 8.3988x; 2.2118x over previous
import functools

import jax
import jax.numpy as jnp
from jax import lax
from jax.experimental import pallas as pl
from jax.experimental.pallas import tpu as pltpu
from jax.experimental.pallas import tpu_sc as plsc

_N_NODES = 10000
_N_EDGES = 320000
_D = 128
_NC = 2
_NS = 16
_NW = _NC * _NS
_CHUNK = 128
_NCHUNKS = _N_EDGES // _CHUNK
_GRP = 8
_NGROUPS = (_NCHUNKS + _GRP - 1) // _GRP
_NGMAX = (_NGROUPS + _NW - 1) // _NW
_NPAD = 10240
_ROWS_PT = _NPAD // _NS
_NHIST = 10240
_L = 16

_mesh = plsc.VectorSubcoreMesh(
    core_axis_name="c", subcore_axis_name="s", num_cores=_NC, num_subcores=_NS)


@functools.partial(
    pl.kernel,
    out_type=[
        jax.ShapeDtypeStruct((_NC, _NPAD, _D), jnp.float32),
        jax.ShapeDtypeStruct((_NW * _NHIST,), jnp.float32),
    ],
    mesh=_mesh,
    scratch_types=[
        pltpu.VMEM_SHARED((_NPAD, _D), jnp.float32),
        pltpu.VMEM((_GRP, _CHUNK), jnp.int32),
        pltpu.VMEM((_CHUNK, _D), jnp.float32),
        pltpu.VMEM((_CHUNK, _D), jnp.float32),
        pltpu.VMEM((_NHIST,), jnp.float32),
        pltpu.SemaphoreType.DMA,
        pltpu.SemaphoreType.DMA,
    ],
    compiler_params=pltpu.CompilerParams(needs_layout_passes=False),
)
def _sc_aggregate(edges_hbm, idx_hbm, zeros_hbm, zeros_h_hbm,
                  sums_out, cnts_out,
                  accum, idx_blk, ebuf0, ebuf1, cnt_v, sem0, sem1):
    cid = lax.axis_index("c")
    sid = lax.axis_index("s")
    wid = cid * _NS + sid
    rbase = pl.multiple_of(sid * _ROWS_PT, _ROWS_PT)
    pltpu.sync_copy(zeros_hbm, accum.at[pl.ds(rbase, _ROWS_PT)])
    pltpu.sync_copy(zeros_h_hbm, cnt_v)
    plsc.subcore_barrier()

    ones16 = jnp.full((_L,), 1.0, jnp.float32)

    def group_body(j, carry):
        g = wid + _NW * j

        @pl.when(g < _NGROUPS)
        def _process_group():
            pltpu.sync_copy(idx_hbm.at[g], idx_blk)
            bufs = (ebuf0, ebuf1)
            sems = (sem0, sem1)

            @pl.when(g * _GRP < _NCHUNKS)
            def _fire_first():
                ebase = pl.multiple_of(g * _GRP * _CHUNK, _CHUNK)
                pltpu.async_copy(edges_hbm.at[pl.ds(ebase, _CHUNK)],
                                 ebuf0, sem0)

            for r in range(_GRP):
                q = g * _GRP + r
                buf, sem = bufs[r % 2], sems[r % 2]
                nbuf, nsem = bufs[(r + 1) % 2], sems[(r + 1) % 2]

                if r + 1 < _GRP:
                    @pl.when(q + 1 < _NCHUNKS)
                    def _fire_next(q=q, nbuf=nbuf, nsem=nsem):
                        ebase = pl.multiple_of((q + 1) * _CHUNK, _CHUNK)
                        pltpu.async_copy(edges_hbm.at[pl.ds(ebase, _CHUNK)],
                                         nbuf, nsem)

                @pl.when(q < _NCHUNKS)
                def _process_chunk(q=q, r=r, buf=buf, sem=sem):
                    ebase = pl.multiple_of(q * _CHUNK, _CHUNK)
                    pltpu.make_async_copy(edges_hbm.at[pl.ds(ebase, _CHUNK)],
                                          buf, sem).wait()
                    pltpu.sync_copy(buf, accum.at[idx_blk.at[r]], add=True)
                    for k in range(_CHUNK // _L):
                        iv = idx_blk[r, pl.ds(k * _L, _L)]
                        plsc.addupdate_scatter(cnt_v, [iv], ones16)

        return carry

    lax.fori_loop(0, _NGMAX, group_body, 0)
    plsc.subcore_barrier()

    pltpu.sync_copy(accum.at[pl.ds(rbase, _ROWS_PT)],
                    sums_out.at[cid, pl.ds(rbase, _ROWS_PT)])
    hbase = pl.multiple_of(wid * _NHIST, _NHIST)
    pltpu.sync_copy(cnt_v, cnts_out.at[pl.ds(hbase, _NHIST)])


_BLK = 1024
_HROWS = _BLK // _D


def _tc_body(s_ref, c_ref, w_ref, b_ref, o_ref):
    s = s_ref[0] + s_ref[1]
    c8 = jnp.sum(c_ref[...], axis=0)
    sel_r = lax.broadcasted_iota(jnp.int32, (_BLK, _HROWS), 0) // _D
    sel = (lax.broadcasted_iota(jnp.int32, (_BLK, _HROWS), 1)
           == sel_r).astype(jnp.float32)
    rep = lax.dot_general(sel, c8, (((1,), (0,)), ((), ())),
                          preferred_element_type=jnp.float32)
    colmask = (lax.broadcasted_iota(jnp.int32, (_BLK, _D), 1)
               == lax.broadcasted_iota(jnp.int32, (_BLK, _D), 0) % _D)
    cnt = jnp.sum(jnp.where(colmask, rep, 0.0), axis=1, keepdims=True)
    mean = s / jnp.maximum(cnt, 1.0)
    acc = lax.dot_general(mean, w_ref[...], (((1,), (1,)), ((), ())),
                          preferred_element_type=jnp.float32)
    o_ref[...] = jnp.maximum(acc + b_ref[...], 0.0)


_tc_update = pl.pallas_call(
    _tc_body,
    grid=((_N_NODES + _BLK - 1) // _BLK,),
    in_specs=[
        pl.BlockSpec((_NC, _BLK, _D), lambda i: (0, i, 0)),
        pl.BlockSpec((_NW, _HROWS, _D), lambda i: (0, i, 0)),
        pl.BlockSpec((_D, _D), lambda i: (0, 0)),
        pl.BlockSpec((1, _D), lambda i: (0, 0)),
    ],
    out_specs=pl.BlockSpec((_BLK, _D), lambda i: (i, 0)),
    out_shape=jax.ShapeDtypeStruct((_N_NODES, _D), jnp.float32),
)


def kernel(edge_data, edge_index, W, b):
    dst = edge_index[1]
    dst_pad = jnp.pad(dst, (0, _NGROUPS * _GRP * _CHUNK - _N_EDGES))
    idx3 = dst_pad.reshape(_NGROUPS, _GRP, _CHUNK)
    zeros = jnp.zeros((_ROWS_PT, _D), jnp.float32)
    zeros_h = jnp.zeros((_NHIST,), jnp.float32)
    sums, cnts = _sc_aggregate(edge_data, idx3, zeros, zeros_h)
    cnts3 = cnts.reshape(_NW, _NHIST // _D, _D)
    return _tc_update(sums, cnts3, W, b.reshape(1, _D))


# speedup vs baseline: 8.4840x; 1.0102x over previous
import functools

import jax
import jax.numpy as jnp
from jax import lax
from jax.experimental import pallas as pl
from jax.experimental.pallas import tpu as pltpu
from jax.experimental.pallas import tpu_sc as plsc

_N_NODES = 10000
_N_EDGES = 320000
_D = 128
_NC = 2
_NS = 16
_NW = _NC * _NS
_CHUNK = 128
_NCHUNKS = _N_EDGES // _CHUNK
_GRP = 8
_NGROUPS = (_NCHUNKS + _GRP - 1) // _GRP
_NGMAX = (_NGROUPS + _NW - 1) // _NW
_NPAD = 10240
_ROWS_PT = _NPAD // _NS
_NHIST = 10240
_L = 16

_mesh = plsc.VectorSubcoreMesh(
    core_axis_name="c", subcore_axis_name="s", num_cores=_NC, num_subcores=_NS)


@functools.partial(
    pl.kernel,
    out_type=[
        jax.ShapeDtypeStruct((_NC, _NPAD, _D), jnp.float32),
        jax.ShapeDtypeStruct((_NW * _NHIST,), jnp.float32),
    ],
    mesh=_mesh,
    scratch_types=[
        pltpu.VMEM_SHARED((_NPAD, _D), jnp.float32),
        pltpu.VMEM((_GRP, _CHUNK), jnp.int32),
        pltpu.VMEM((_CHUNK, _D), jnp.float32),
        pltpu.VMEM((_CHUNK, _D), jnp.float32),
        pltpu.VMEM((_NHIST,), jnp.float32),
        pltpu.SemaphoreType.DMA,
        pltpu.SemaphoreType.DMA,
        pltpu.SemaphoreType.DMA,
        pltpu.SemaphoreType.DMA,
    ],
    compiler_params=pltpu.CompilerParams(needs_layout_passes=False),
)
def _sc_aggregate(edges_hbm, idx_hbm, zeros_hbm, zeros_h_hbm,
                  sums_out, cnts_out,
                  accum, idx_blk, ebuf0, ebuf1, cnt_v,
                  sem0, sem1, ssem0, ssem1):
    cid = lax.axis_index("c")
    sid = lax.axis_index("s")
    wid = cid * _NS + sid
    rbase = pl.multiple_of(sid * _ROWS_PT, _ROWS_PT)
    pltpu.sync_copy(zeros_hbm, accum.at[pl.ds(rbase, _ROWS_PT)])
    pltpu.sync_copy(zeros_h_hbm, cnt_v)
    plsc.subcore_barrier()

    ones16 = jnp.full((_L,), 1.0, jnp.float32)

    def group_body(j, carry):
        g = wid + _NW * j

        @pl.when(g < _NGROUPS)
        def _process_group():
            pltpu.sync_copy(idx_hbm.at[g], idx_blk)
            bufs = (ebuf0, ebuf1)
            esems = (sem0, sem1)
            ssems = (ssem0, ssem1)

            def _edma(q, buf, sem):
                ebase = pl.multiple_of(q * _CHUNK, _CHUNK)
                return pltpu.make_async_copy(
                    edges_hbm.at[pl.ds(ebase, _CHUNK)], buf, sem)

            def _scat(r, buf, sem):
                return pltpu.make_async_copy(
                    buf, accum.at[idx_blk.at[r]], sem)

            @pl.when(g * _GRP < _NCHUNKS)
            def _fire_first():
                _edma(g * _GRP, ebuf0, sem0).start()

            @pl.when(g * _GRP + 1 < _NCHUNKS)
            def _fire_second():
                _edma(g * _GRP + 1, ebuf1, sem1).start()

            for r in range(_GRP):
                q = g * _GRP + r
                buf, esem, ssem = bufs[r % 2], esems[r % 2], ssems[r % 2]

                @pl.when(q < _NCHUNKS)
                def _process_chunk(q=q, r=r, buf=buf, esem=esem, ssem=ssem):
                    _edma(q, buf, esem).wait()
                    _scat(r, buf, ssem).start(add=True)
                    for k in range(_CHUNK // _L):
                        iv = idx_blk[r, pl.ds(k * _L, _L)]
                        plsc.addupdate_scatter(cnt_v, [iv], ones16)
                    if r + 2 < _GRP:
                        @pl.when(q + 2 < _NCHUNKS)
                        def _refill():
                            _scat(r, buf, ssem).wait()
                            _edma(q + 2, buf, esem).start()

                        @pl.when(q + 2 >= _NCHUNKS)
                        def _drain_only():
                            _scat(r, buf, ssem).wait()
                    else:
                        _scat(r, buf, ssem).wait()

        return carry

    lax.fori_loop(0, _NGMAX, group_body, 0)
    plsc.subcore_barrier()

    pltpu.sync_copy(accum.at[pl.ds(rbase, _ROWS_PT)],
                    sums_out.at[cid, pl.ds(rbase, _ROWS_PT)])
    hbase = pl.multiple_of(wid * _NHIST, _NHIST)
    pltpu.sync_copy(cnt_v, cnts_out.at[pl.ds(hbase, _NHIST)])


_BLK = 1024
_HROWS = _BLK // _D


def _tc_body(s_ref, c_ref, w_ref, b_ref, o_ref):
    s = s_ref[0] + s_ref[1]
    c8 = jnp.sum(c_ref[...], axis=0)
    sel_r = lax.broadcasted_iota(jnp.int32, (_BLK, _HROWS), 0) // _D
    sel = (lax.broadcasted_iota(jnp.int32, (_BLK, _HROWS), 1)
           == sel_r).astype(jnp.float32)
    rep = lax.dot_general(sel, c8, (((1,), (0,)), ((), ())),
                          preferred_element_type=jnp.float32)
    colmask = (lax.broadcasted_iota(jnp.int32, (_BLK, _D), 1)
               == lax.broadcasted_iota(jnp.int32, (_BLK, _D), 0) % _D)
    cnt = jnp.sum(jnp.where(colmask, rep, 0.0), axis=1, keepdims=True)
    mean = s / jnp.maximum(cnt, 1.0)
    acc = lax.dot_general(mean, w_ref[...], (((1,), (1,)), ((), ())),
                          preferred_element_type=jnp.float32)
    o_ref[...] = jnp.maximum(acc + b_ref[...], 0.0)


_tc_update = pl.pallas_call(
    _tc_body,
    grid=((_N_NODES + _BLK - 1) // _BLK,),
    in_specs=[
        pl.BlockSpec((_NC, _BLK, _D), lambda i: (0, i, 0)),
        pl.BlockSpec((_NW, _HROWS, _D), lambda i: (0, i, 0)),
        pl.BlockSpec((_D, _D), lambda i: (0, 0)),
        pl.BlockSpec((1, _D), lambda i: (0, 0)),
    ],
    out_specs=pl.BlockSpec((_BLK, _D), lambda i: (i, 0)),
    out_shape=jax.ShapeDtypeStruct((_N_NODES, _D), jnp.float32),
)


def kernel(edge_data, edge_index, W, b):
    dst = edge_index[1]
    dst_pad = jnp.pad(dst, (0, _NGROUPS * _GRP * _CHUNK - _N_EDGES))
    idx3 = dst_pad.reshape(_NGROUPS, _GRP, _CHUNK)
    zeros = jnp.zeros((_ROWS_PT, _D), jnp.float32)
    zeros_h = jnp.zeros((_NHIST,), jnp.float32)
    sums, cnts = _sc_aggregate(edge_data, idx3, zeros, zeros_h)
    cnts3 = cnts.reshape(_NW, _NHIST // _D, _D)
    return _tc_update(sums, cnts3, W, b.reshape(1, _D))


# speedup vs baseline: 8.7733x; 1.0341x over previous
import functools

import jax
import jax.numpy as jnp
from jax import lax
from jax.experimental import pallas as pl
from jax.experimental.pallas import tpu as pltpu
from jax.experimental.pallas import tpu_sc as plsc

_N_NODES = 10000
_N_EDGES = 320000
_D = 128
_NC = 2
_NS = 16
_NW = _NC * _NS
_CHUNK = 128
_NCHUNKS = _N_EDGES // _CHUNK
_GRP = 8
_NGROUPS = _NCHUNKS // _GRP
_NTAIL = _NCHUNKS - _NGROUPS * _GRP
_NGMAX = (_NGROUPS + _NW - 1) // _NW
_NPAD = 10240
_ROWS_PT = _NPAD // _NS
_NHIST = 10240
_L = 16

_mesh = plsc.VectorSubcoreMesh(
    core_axis_name="c", subcore_axis_name="s", num_cores=_NC, num_subcores=_NS)


@functools.partial(
    pl.kernel,
    out_type=[
        jax.ShapeDtypeStruct((_NC, _NPAD, _D), jnp.float32),
        jax.ShapeDtypeStruct((_NW * _NHIST,), jnp.float32),
    ],
    mesh=_mesh,
    scratch_types=[
        pltpu.VMEM_SHARED((_NPAD, _D), jnp.float32),
        pltpu.VMEM((_GRP, _CHUNK), jnp.int32),
        pltpu.VMEM((1, _CHUNK), jnp.int32),
        pltpu.VMEM((_CHUNK, _D), jnp.float32),
        pltpu.VMEM((_CHUNK, _D), jnp.float32),
        pltpu.VMEM((_NHIST,), jnp.float32),
        pltpu.SemaphoreType.DMA,
        pltpu.SemaphoreType.DMA,
        pltpu.SemaphoreType.DMA,
        pltpu.SemaphoreType.DMA,
    ],
    compiler_params=pltpu.CompilerParams(needs_layout_passes=False),
)
def _sc_aggregate(edges_hbm, idx_hbm, tail_hbm, zeros_hbm, zeros_h_hbm,
                  sums_out, cnts_out,
                  accum, idx_blk, tail_blk, ebuf0, ebuf1, cnt_v,
                  sem0, sem1, ssem0, ssem1):
    cid = lax.axis_index("c")
    sid = lax.axis_index("s")
    wid = cid * _NS + sid
    rbase = pl.multiple_of(sid * _ROWS_PT, _ROWS_PT)
    pltpu.sync_copy(zeros_hbm, accum.at[pl.ds(rbase, _ROWS_PT)])
    pltpu.sync_copy(zeros_h_hbm, cnt_v)
    plsc.subcore_barrier()

    ones16 = jnp.full((_L,), 1.0, jnp.float32)

    def group_body(j, carry):
        g = wid + _NW * j

        @pl.when(g < _NGROUPS)
        def _process_group():
            pltpu.sync_copy(idx_hbm.at[g], idx_blk)
            bufs = (ebuf0, ebuf1)
            esems = (sem0, sem1)
            ssems = (ssem0, ssem1)

            def _edma(q, buf, sem):
                ebase = pl.multiple_of(q * _CHUNK, _CHUNK)
                return pltpu.make_async_copy(
                    edges_hbm.at[pl.ds(ebase, _CHUNK)], buf, sem)

            def _scat(r, buf, sem):
                return pltpu.make_async_copy(
                    buf, accum.at[idx_blk.at[r]], sem)

            _edma(g * _GRP, ebuf0, sem0).start()
            _edma(g * _GRP + 1, ebuf1, sem1).start()

            for r in range(_GRP):
                q = g * _GRP + r
                buf, esem, ssem = bufs[r % 2], esems[r % 2], ssems[r % 2]
                _edma(q, buf, esem).wait()
                _scat(r, buf, ssem).start(add=True)
                for k in range(_CHUNK // _L):
                    iv = idx_blk[r, pl.ds(k * _L, _L)]
                    plsc.addupdate_scatter(cnt_v, [iv], ones16)
                _scat(r, buf, ssem).wait()
                if r + 2 < _GRP:
                    _edma(q + 2, buf, esem).start()

        return carry

    lax.fori_loop(0, _NGMAX, group_body, 0)

    @pl.when(wid < _NTAIL)
    def _tail():
        pltpu.sync_copy(tail_hbm.at[wid], tail_blk)
        ebase = pl.multiple_of((_NGROUPS * _GRP + wid) * _CHUNK, _CHUNK)
        pltpu.sync_copy(edges_hbm.at[pl.ds(ebase, _CHUNK)], ebuf0)
        pltpu.sync_copy(ebuf0, accum.at[tail_blk.at[0]], add=True)
        for k in range(_CHUNK // _L):
            iv = tail_blk[0, pl.ds(k * _L, _L)]
            plsc.addupdate_scatter(cnt_v, [iv], ones16)

    plsc.subcore_barrier()

    pltpu.sync_copy(accum.at[pl.ds(rbase, _ROWS_PT)],
                    sums_out.at[cid, pl.ds(rbase, _ROWS_PT)])
    hbase = pl.multiple_of(wid * _NHIST, _NHIST)
    pltpu.sync_copy(cnt_v, cnts_out.at[pl.ds(hbase, _NHIST)])


_BLK = 1024
_HROWS = _BLK // _D


def _tc_body(s_ref, c_ref, w_ref, b_ref, o_ref):
    s = s_ref[0] + s_ref[1]
    c8 = jnp.sum(c_ref[...], axis=0)
    sel_r = lax.broadcasted_iota(jnp.int32, (_BLK, _HROWS), 0) // _D
    sel = (lax.broadcasted_iota(jnp.int32, (_BLK, _HROWS), 1)
           == sel_r).astype(jnp.float32)
    rep = lax.dot_general(sel, c8, (((1,), (0,)), ((), ())),
                          preferred_element_type=jnp.float32)
    colmask = (lax.broadcasted_iota(jnp.int32, (_BLK, _D), 1)
               == lax.broadcasted_iota(jnp.int32, (_BLK, _D), 0) % _D)
    cnt = jnp.sum(jnp.where(colmask, rep, 0.0), axis=1, keepdims=True)
    mean = s / jnp.maximum(cnt, 1.0)
    acc = lax.dot_general(mean, w_ref[...], (((1,), (1,)), ((), ())),
                          preferred_element_type=jnp.float32)
    o_ref[...] = jnp.maximum(acc + b_ref[...], 0.0)


_tc_update = pl.pallas_call(
    _tc_body,
    grid=((_N_NODES + _BLK - 1) // _BLK,),
    in_specs=[
        pl.BlockSpec((_NC, _BLK, _D), lambda i: (0, i, 0)),
        pl.BlockSpec((_NW, _HROWS, _D), lambda i: (0, i, 0)),
        pl.BlockSpec((_D, _D), lambda i: (0, 0)),
        pl.BlockSpec((1, _D), lambda i: (0, 0)),
    ],
    out_specs=pl.BlockSpec((_BLK, _D), lambda i: (i, 0)),
    out_shape=jax.ShapeDtypeStruct((_N_NODES, _D), jnp.float32),
)


def kernel(edge_data, edge_index, W, b):
    dst = edge_index[1]
    nfull = _NGROUPS * _GRP * _CHUNK
    idx3 = dst[:nfull].reshape(_NGROUPS, _GRP, _CHUNK)
    tail = dst[nfull:].reshape(_NTAIL, 1, _CHUNK)
    zeros = jnp.zeros((_ROWS_PT, _D), jnp.float32)
    zeros_h = jnp.zeros((_NHIST,), jnp.float32)
    sums, cnts = _sc_aggregate(edge_data, idx3, tail, zeros, zeros_h)
    cnts3 = cnts.reshape(_NW, _NHIST // _D, _D)
    return _tc_update(sums, cnts3, W, b.reshape(1, _D))


# speedup vs baseline: 9.0569x; 1.0323x over previous
import functools

import jax
import jax.numpy as jnp
from jax import lax
from jax.experimental import pallas as pl
from jax.experimental.pallas import tpu as pltpu
from jax.experimental.pallas import tpu_sc as plsc

_N_NODES = 10000
_N_EDGES = 320000
_D = 128
_NC = 2
_NS = 16
_NW = _NC * _NS
_CHUNK = 128
_NCHUNKS = _N_EDGES // _CHUNK
_GRP = 8
_NGROUPS = _NCHUNKS // _GRP
_NTAIL = _NCHUNKS - _NGROUPS * _GRP
_NGMAX = (_NGROUPS + _NW - 1) // _NW
_NPAD = 10240
_ROWS_PT = _NPAD // _NS
_NHIST = 10240
_L = 16

_mesh = plsc.VectorSubcoreMesh(
    core_axis_name="c", subcore_axis_name="s", num_cores=_NC, num_subcores=_NS)


@functools.partial(
    pl.kernel,
    out_type=[
        jax.ShapeDtypeStruct((_NC, _NPAD, _D), jnp.float32),
        jax.ShapeDtypeStruct((_NW * _NHIST,), jnp.float32),
    ],
    mesh=_mesh,
    scratch_types=[
        pltpu.VMEM_SHARED((_NPAD, _D), jnp.float32),
        pltpu.VMEM((_GRP, _CHUNK), jnp.int32),
        pltpu.VMEM((_GRP, _CHUNK), jnp.int32),
        pltpu.VMEM((1, _CHUNK), jnp.int32),
        pltpu.VMEM((_CHUNK, _D), jnp.float32),
        pltpu.VMEM((_CHUNK, _D), jnp.float32),
        pltpu.VMEM((_NHIST,), jnp.float32),
        pltpu.SemaphoreType.DMA,
        pltpu.SemaphoreType.DMA,
        pltpu.SemaphoreType.DMA,
        pltpu.SemaphoreType.DMA,
        pltpu.SemaphoreType.DMA,
        pltpu.SemaphoreType.DMA,
    ],
    compiler_params=pltpu.CompilerParams(needs_layout_passes=False),
)
def _sc_aggregate(edges_hbm, idx_hbm, tail_hbm, zeros_hbm, zeros_h_hbm,
                  sums_out, cnts_out,
                  accum, idx_blkA, idx_blkB, tail_blk, ebuf0, ebuf1, cnt_v,
                  sem0, sem1, ssem0, ssem1, isemA, isemB):
    cid = lax.axis_index("c")
    sid = lax.axis_index("s")
    wid = cid * _NS + sid
    rbase = pl.multiple_of(sid * _ROWS_PT, _ROWS_PT)
    pltpu.sync_copy(zeros_hbm, accum.at[pl.ds(rbase, _ROWS_PT)])
    pltpu.sync_copy(zeros_h_hbm, cnt_v)
    plsc.subcore_barrier()

    ones16 = jnp.full((_L,), 1.0, jnp.float32)
    bufs = (ebuf0, ebuf1)
    esems = (sem0, sem1)
    ssems = (ssem0, ssem1)

    def _edma(q, buf, sem):
        ebase = pl.multiple_of(q * _CHUNK, _CHUNK)
        return pltpu.make_async_copy(
            edges_hbm.at[pl.ds(ebase, _CHUNK)], buf, sem)

    def _idma(g, blk, sem):
        return pltpu.make_async_copy(idx_hbm.at[g], blk, sem)

    def _process_group(g, idx_blk):
        def _scat(r, buf, sem):
            return pltpu.make_async_copy(buf, accum.at[idx_blk.at[r]], sem)

        _edma(g * _GRP, ebuf0, sem0).start()
        _edma(g * _GRP + 1, ebuf1, sem1).start()

        for r in range(_GRP):
            q = g * _GRP + r
            buf, esem, ssem = bufs[r % 2], esems[r % 2], ssems[r % 2]
            _edma(q, buf, esem).wait()
            _scat(r, buf, ssem).start(add=True)
            for k in range(_CHUNK // _L):
                iv = idx_blk[r, pl.ds(k * _L, _L)]
                plsc.addupdate_scatter(cnt_v, [iv], ones16)
            _scat(r, buf, ssem).wait()
            if r + 2 < _GRP:
                _edma(q + 2, buf, esem).start()

    _idma(wid, idx_blkA, isemA).start()

    def pair_body(u, carry):
        gA = wid + _NW * 2 * u
        gB = gA + _NW

        @pl.when(gB < _NGROUPS)
        def _fire_b():
            _idma(gB, idx_blkB, isemB).start()

        @pl.when(gA < _NGROUPS)
        def _do_a():
            _idma(gA, idx_blkA, isemA).wait()
            _process_group(gA, idx_blkA)

        gA2 = gA + 2 * _NW

        @pl.when(gA2 < _NGROUPS)
        def _fire_a_next():
            _idma(gA2, idx_blkA, isemA).start()

        @pl.when(gB < _NGROUPS)
        def _do_b():
            _idma(gB, idx_blkB, isemB).wait()
            _process_group(gB, idx_blkB)

        return carry

    lax.fori_loop(0, (_NGMAX + 1) // 2, pair_body, 0)

    @pl.when(wid < _NTAIL)
    def _tail():
        pltpu.sync_copy(tail_hbm.at[wid], tail_blk)
        ebase = pl.multiple_of((_NGROUPS * _GRP + wid) * _CHUNK, _CHUNK)
        pltpu.sync_copy(edges_hbm.at[pl.ds(ebase, _CHUNK)], ebuf0)
        pltpu.sync_copy(ebuf0, accum.at[tail_blk.at[0]], add=True)
        for k in range(_CHUNK // _L):
            iv = tail_blk[0, pl.ds(k * _L, _L)]
            plsc.addupdate_scatter(cnt_v, [iv], ones16)

    plsc.subcore_barrier()

    pltpu.sync_copy(accum.at[pl.ds(rbase, _ROWS_PT)],
                    sums_out.at[cid, pl.ds(rbase, _ROWS_PT)])
    hbase = pl.multiple_of(wid * _NHIST, _NHIST)
    pltpu.sync_copy(cnt_v, cnts_out.at[pl.ds(hbase, _NHIST)])


_BLK = 1024
_HROWS = _BLK // _D


def _tc_body(s_ref, c_ref, w_ref, b_ref, o_ref):
    s = s_ref[0] + s_ref[1]
    c8 = jnp.sum(c_ref[...], axis=0)
    sel_r = lax.broadcasted_iota(jnp.int32, (_BLK, _HROWS), 0) // _D
    sel = (lax.broadcasted_iota(jnp.int32, (_BLK, _HROWS), 1)
           == sel_r).astype(jnp.float32)
    rep = lax.dot_general(sel, c8, (((1,), (0,)), ((), ())),
                          preferred_element_type=jnp.float32)
    colmask = (lax.broadcasted_iota(jnp.int32, (_BLK, _D), 1)
               == lax.broadcasted_iota(jnp.int32, (_BLK, _D), 0) % _D)
    cnt = jnp.sum(jnp.where(colmask, rep, 0.0), axis=1, keepdims=True)
    mean = s / jnp.maximum(cnt, 1.0)
    acc = lax.dot_general(mean, w_ref[...], (((1,), (1,)), ((), ())),
                          preferred_element_type=jnp.float32)
    o_ref[...] = jnp.maximum(acc + b_ref[...], 0.0)


_tc_update = pl.pallas_call(
    _tc_body,
    grid=((_N_NODES + _BLK - 1) // _BLK,),
    in_specs=[
        pl.BlockSpec((_NC, _BLK, _D), lambda i: (0, i, 0)),
        pl.BlockSpec((_NW, _HROWS, _D), lambda i: (0, i, 0)),
        pl.BlockSpec((_D, _D), lambda i: (0, 0)),
        pl.BlockSpec((1, _D), lambda i: (0, 0)),
    ],
    out_specs=pl.BlockSpec((_BLK, _D), lambda i: (i, 0)),
    out_shape=jax.ShapeDtypeStruct((_N_NODES, _D), jnp.float32),
)


def kernel(edge_data, edge_index, W, b):
    dst = edge_index[1]
    nfull = _NGROUPS * _GRP * _CHUNK
    idx3 = dst[:nfull].reshape(_NGROUPS, _GRP, _CHUNK)
    tail = dst[nfull:].reshape(_NTAIL, 1, _CHUNK)
    zeros = jnp.zeros((_ROWS_PT, _D), jnp.float32)
    zeros_h = jnp.zeros((_NHIST,), jnp.float32)
    sums, cnts = _sc_aggregate(edge_data, idx3, tail, zeros, zeros_h)
    cnts3 = cnts.reshape(_NW, _NHIST // _D, _D)
    return _tc_update(sums, cnts3, W, b.reshape(1, _D))
